# baseline (speedup 1.0000x reference)
import jax
import jax.numpy as jnp
from jax import lax
from jax.experimental import pallas as pl
from jax.experimental.pallas import tpu as pltpu
from jax.experimental.pallas import tpu_sc as plsc

_B = 16384
_S = 26
_ROWS = _B * _S
_D = 32
_V = 1000000
_NC = 2
_NS = 16
_NW = _NC * _NS
_RPW = _ROWS // _NW
_NCHUNK = 8
_C = _RPW // _NCHUNK

_VMAIN = 999936
_VTAIL = _V - _VMAIN
_LB = 1024
_PB = _LB // 4
_NFULL = _V // _LB
_TAIL = _V - _NFULL * _LB
_PTAIL = _TAIL // 4
_KMAX = (_NFULL + _NW - 1) // _NW


def _transform_body(wt_hbm, wp_hbm, vin, vout, d_lo_ref, d_hi_ref):
    wid = lax.axis_index("s") * _NC + lax.axis_index("c")
    iota = lax.iota(jnp.int32, 16)
    d_lo_ref[...] = iota
    d_hi_ref[...] = iota + 16

    def do_block(l0, L, P):
        l0 = pl.multiple_of(l0, 128)
        pltpu.sync_copy(wt_hbm.at[:, pl.ds(l0, L)], vin.at[:, pl.ds(0, L)])

        def body(p, carry):
            e0 = p * 4
            d_lo = d_lo_ref[...]
            d_hi = d_hi_ref[...]
            for g in range(8):
                q = g >> 1
                dv = d_lo if (g & 1) == 0 else d_hi
                ev = jnp.full((16,), 0, jnp.int32) + (e0 + q)
                vals = plsc.load_gather(vin, [dv, ev])
                vout[p, pl.ds(q * 32 + (g & 1) * 16, 16)] = vals
            return carry
        lax.fori_loop(0, P, body, 0)
        pltpu.sync_copy(vout.at[pl.ds(0, P), :],
                        wp_hbm.at[pl.ds(pl.multiple_of(l0 // 4, 8), P), :])

    def step(k, carry):
        b = wid + k * _NW

        @pl.when(b < _NFULL)
        def _():
            do_block(b * _LB, _LB, _PB)
        return carry
    lax.fori_loop(0, _KMAX, step, 0)

    @pl.when(wid == 16)
    def _():
        do_block(_NFULL * _LB, 512, 128)


def _gather_body(idx_hbm, table_hbm, tail_hbm, out_hbm, idx_v, rows_v, tail_v,
                 sem):
    wid = lax.axis_index("s") * _NC + lax.axis_index("c")
    base = wid * _RPW
    pltpu.sync_copy(tail_hbm, tail_v)
    iota = lax.iota(jnp.int32, 16)
    for i in range(_NCHUNK):
        off = base + i * _C
        pltpu.sync_copy(idx_hbm.at[pl.ds(off, _C)], idx_v)
        pltpu.async_copy(table_hbm.at[idx_v], rows_v, sem).wait()
        def fix(v, carry):
            ids = idx_v[pl.ds(v * 16, 16)]
            m = ids >= _VMAIN
            npc = plsc.all_reduce_population_count(m)

            @pl.when(npc[0] > 0)
            def _():
                t = jnp.clip(ids - _VMAIN, 0, _VTAIL - 1)
                rowpos = v * 16 + iota
                for c in range(_D):
                    cvec = jnp.full((16,), 0, jnp.int32) + c
                    vals = plsc.load_gather(tail_v, [t, cvec])
                    plsc.store_scatter(rows_v, [rowpos, cvec], vals, mask=m)
            return carry
        lax.fori_loop(0, _C // 16, fix, 0)
        pltpu.sync_copy(rows_v, out_hbm.at[pl.ds(off, _C)])


_MESH = plsc.VectorSubcoreMesh(core_axis_name="c", subcore_axis_name="s")


@jax.jit
def kernel(token_ids, weights):
    wt = weights.T
    transform = pl.kernel(
        _transform_body,
        mesh=_MESH,
        out_type=jax.ShapeDtypeStruct((_V // 4, 128), jnp.float32),
        compiler_params=pltpu.CompilerParams(use_tc_tiling_on_sc=True, needs_layout_passes=False),
        scratch_types=[
            pltpu.VMEM((_D, _LB), jnp.float32),
            pltpu.VMEM((_PB, 128), jnp.float32),
            pltpu.VMEM((16,), jnp.int32),
            pltpu.VMEM((16,), jnp.int32),
        ],
    )
    table = transform(wt).reshape(_V, _D)
    tail = weights[_VMAIN:, :]
    flat_ids = token_ids.reshape(_ROWS).astype(jnp.int32)
    run = pl.kernel(
        _gather_body,
        mesh=_MESH,
        out_type=jax.ShapeDtypeStruct((_ROWS, _D), jnp.float32),
        compiler_params=pltpu.CompilerParams(use_tc_tiling_on_sc=False, needs_layout_passes=False),
        scratch_types=[
            pltpu.VMEM((_C,), jnp.int32),
            pltpu.VMEM((_C, _D), jnp.float32),
            pltpu.VMEM((_VTAIL, _D), jnp.float32),
            pltpu.SemaphoreType.DMA,
        ],
    )
    out = run(flat_ids, table, tail)
    return out.reshape(_B, _S, _D)


# speedup vs baseline: 1.5110x; 1.5110x over previous
import jax
import jax.numpy as jnp
from jax import lax
from jax.experimental import pallas as pl
from jax.experimental.pallas import tpu as pltpu
from jax.experimental.pallas import tpu_sc as plsc

_B = 16384
_S = 26
_ROWS = _B * _S
_D = 32
_V = 1000000
_NC = 2
_NS = 16
_NW = _NC * _NS
_RPW = _ROWS // _NW
_NCHUNK = 8
_C = _RPW // _NCHUNK

_VMAIN = 999936
_VTAIL = _V - _VMAIN
_LB = 1024
_PB = _LB // 4
_NFULL = _V // _LB
_TAIL = _V - _NFULL * _LB
_PTAIL = _TAIL // 4
_KMAX = (_NFULL + _NW - 1) // _NW


def _transform_body(wt_hbm, wp_hbm, vin, vout):
    wid = lax.axis_index("s") * _NC + lax.axis_index("c")
    iota = lax.iota(jnp.int32, 16)
    d_lo = iota
    d_hi = iota + 16
    zeros = jnp.full((16,), 0, jnp.int32)

    def do_block(l0, L, P):
        l0 = pl.multiple_of(l0, 128)
        pltpu.sync_copy(wt_hbm.at[:, pl.ds(l0, L)], vin.at[:, pl.ds(0, L)])

        @plsc.parallel_loop(0, P, unroll=8)
        def body(p):
            ev0 = zeros + p * 4
            for g in range(8):
                q = g >> 1
                dv = d_lo if (g & 1) == 0 else d_hi
                vals = plsc.load_gather(vin, [dv, ev0 + q])
                vout[p, pl.ds(q * 32 + (g & 1) * 16, 16)] = vals
        pltpu.sync_copy(vout.at[pl.ds(0, P), :],
                        wp_hbm.at[pl.ds(pl.multiple_of(l0 // 4, 8), P), :])

    def step(k, carry):
        b = wid + k * _NW

        @pl.when(b < _NFULL)
        def _():
            do_block(b * _LB, _LB, _PB)
        return carry
    lax.fori_loop(0, _KMAX, step, 0)

    @pl.when(wid == 16)
    def _():
        do_block(_NFULL * _LB, 512, 128)


def _gather_body(idx_hbm, table_hbm, tail_hbm, out_hbm, idx_v, rows_v, tail_v,
                 sem):
    wid = lax.axis_index("s") * _NC + lax.axis_index("c")
    base = wid * _RPW
    pltpu.sync_copy(tail_hbm, tail_v)
    iota = lax.iota(jnp.int32, 16)
    for i in range(_NCHUNK):
        off = base + i * _C
        pltpu.sync_copy(idx_hbm.at[pl.ds(off, _C)], idx_v)
        pltpu.async_copy(table_hbm.at[idx_v], rows_v, sem).wait()
        def fix(v, carry):
            ids = idx_v[pl.ds(v * 16, 16)]
            m = ids >= _VMAIN
            npc = plsc.all_reduce_population_count(m)

            @pl.when(npc[0] > 0)
            def _():
                t = jnp.clip(ids - _VMAIN, 0, _VTAIL - 1)
                rowpos = v * 16 + iota
                for c in range(_D):
                    cvec = jnp.full((16,), 0, jnp.int32) + c
                    vals = plsc.load_gather(tail_v, [t, cvec])
                    plsc.store_scatter(rows_v, [rowpos, cvec], vals, mask=m)
            return carry
        lax.fori_loop(0, _C // 16, fix, 0)
        pltpu.sync_copy(rows_v, out_hbm.at[pl.ds(off, _C)])


_MESH = plsc.VectorSubcoreMesh(core_axis_name="c", subcore_axis_name="s")


@jax.jit
def kernel(token_ids, weights):
    wt = weights.T
    transform = pl.kernel(
        _transform_body,
        mesh=_MESH,
        out_type=jax.ShapeDtypeStruct((_V // 4, 128), jnp.float32),
        compiler_params=pltpu.CompilerParams(use_tc_tiling_on_sc=True, needs_layout_passes=False),
        scratch_types=[
            pltpu.VMEM((_D, _LB), jnp.float32),
            pltpu.VMEM((_PB, 128), jnp.float32),
        ],
    )
    table = transform(wt).reshape(_V, _D)
    tail = weights[_VMAIN:, :]
    flat_ids = token_ids.reshape(_ROWS).astype(jnp.int32)
    run = pl.kernel(
        _gather_body,
        mesh=_MESH,
        out_type=jax.ShapeDtypeStruct((_ROWS, _D), jnp.float32),
        compiler_params=pltpu.CompilerParams(use_tc_tiling_on_sc=False, needs_layout_passes=False),
        scratch_types=[
            pltpu.VMEM((_C,), jnp.int32),
            pltpu.VMEM((_C, _D), jnp.float32),
            pltpu.VMEM((_VTAIL, _D), jnp.float32),
            pltpu.SemaphoreType.DMA,
        ],
    )
    out = run(flat_ids, table, tail)
    return out.reshape(_B, _S, _D)


# speedup vs baseline: 2.4959x; 1.6518x over previous
import jax
import jax.numpy as jnp
from jax import lax
from jax.experimental import pallas as pl
from jax.experimental.pallas import tpu as pltpu
from jax.experimental.pallas import tpu_sc as plsc

_B = 16384
_S = 26
_ROWS = _B * _S
_D = 32
_V = 1000000
_NC = 2
_NS = 16
_NW = _NC * _NS
_RPW = _ROWS // _NW
_NCHUNK = 8
_C = _RPW // _NCHUNK

_VMAIN = 999936
_VTAIL = _V - _VMAIN
_LB = 1024
_PB = _LB // 4
_NFULL = _V // _LB
_TAIL = _V - _NFULL * _LB
_PTAIL = _TAIL // 4
_KMAX = (_NFULL + _NW - 1) // _NW


def _transform_body(wt_hbm, wp_hbm, vin, vout):
    wid = lax.axis_index("s") * _NC + lax.axis_index("c")
    iota = lax.iota(jnp.int32, 16)
    d_lo = iota
    d_hi = iota + 16
    zeros = jnp.full((16,), 0, jnp.int32)

    def do_block(l0, L, P):
        l0 = pl.multiple_of(l0, 128)
        pltpu.sync_copy(wt_hbm.at[:, pl.ds(l0, L)], vin.at[:, pl.ds(0, L)])

        @plsc.parallel_loop(0, L, unroll=4)
        def body(e0):
            ev = (zeros + e0 + d_lo) & (L - 1)
            pv = ev >> 2
            jb = (ev & 3) * 32
            for h in range(2):
                dv = d_lo if h == 0 else d_hi
                vals = plsc.load_gather(vin, [dv, ev])
                plsc.store_scatter(vout, [pv, jb + dv], vals)
        pltpu.sync_copy(vout.at[pl.ds(0, P), :],
                        wp_hbm.at[pl.ds(pl.multiple_of(l0 // 4, 8), P), :])

    def step(k, carry):
        b = wid + k * _NW

        @pl.when(b < _NFULL)
        def _():
            do_block(b * _LB, _LB, _PB)
        return carry
    lax.fori_loop(0, _KMAX, step, 0)

    @pl.when(wid == 16)
    def _():
        do_block(_NFULL * _LB, 512, 128)


def _gather_body(idx_hbm, table_hbm, tail_hbm, out_hbm, idx_v, rows_v, tail_v,
                 sem):
    wid = lax.axis_index("s") * _NC + lax.axis_index("c")
    base = wid * _RPW
    pltpu.sync_copy(tail_hbm, tail_v)
    iota = lax.iota(jnp.int32, 16)
    for i in range(_NCHUNK):
        off = base + i * _C
        pltpu.sync_copy(idx_hbm.at[pl.ds(off, _C)], idx_v)
        pltpu.async_copy(table_hbm.at[idx_v], rows_v, sem).wait()
        def fix(v, carry):
            ids = idx_v[pl.ds(v * 16, 16)]
            m = ids >= _VMAIN
            npc = plsc.all_reduce_population_count(m)

            @pl.when(npc[0] > 0)
            def _():
                t = jnp.clip(ids - _VMAIN, 0, _VTAIL - 1)
                rowpos = v * 16 + iota
                for c in range(_D):
                    cvec = jnp.full((16,), 0, jnp.int32) + c
                    vals = plsc.load_gather(tail_v, [t, cvec])
                    plsc.store_scatter(rows_v, [rowpos, cvec], vals, mask=m)
            return carry
        lax.fori_loop(0, _C // 16, fix, 0)
        pltpu.sync_copy(rows_v, out_hbm.at[pl.ds(off, _C)])


_MESH = plsc.VectorSubcoreMesh(core_axis_name="c", subcore_axis_name="s")


@jax.jit
def kernel(token_ids, weights):
    wt = weights.T
    transform = pl.kernel(
        _transform_body,
        mesh=_MESH,
        out_type=jax.ShapeDtypeStruct((_V // 4, 128), jnp.float32),
        compiler_params=pltpu.CompilerParams(use_tc_tiling_on_sc=True, needs_layout_passes=False),
        scratch_types=[
            pltpu.VMEM((_D, _LB), jnp.float32),
            pltpu.VMEM((_PB, 128), jnp.float32),
        ],
    )
    table = transform(wt).reshape(_V, _D)
    tail = weights[_VMAIN:, :]
    flat_ids = token_ids.reshape(_ROWS).astype(jnp.int32)
    run = pl.kernel(
        _gather_body,
        mesh=_MESH,
        out_type=jax.ShapeDtypeStruct((_ROWS, _D), jnp.float32),
        compiler_params=pltpu.CompilerParams(use_tc_tiling_on_sc=False, needs_layout_passes=False),
        scratch_types=[
            pltpu.VMEM((_C,), jnp.int32),
            pltpu.VMEM((_C, _D), jnp.float32),
            pltpu.VMEM((_VTAIL, _D), jnp.float32),
            pltpu.SemaphoreType.DMA,
        ],
    )
    out = run(flat_ids, table, tail)
    return out.reshape(_B, _S, _D)


# speedup vs baseline: 3.3439x; 1.3398x over previous
import jax
import jax.numpy as jnp
from jax import lax
from jax.experimental import pallas as pl
from jax.experimental.pallas import tpu as pltpu
from jax.experimental.pallas import tpu_sc as plsc

_B = 16384
_S = 26
_ROWS = _B * _S
_D = 32
_V = 1000000
_NC = 2
_NS = 16
_NW = _NC * _NS
_RPW = _ROWS // _NW
_NCHUNK = 8
_C = _RPW // _NCHUNK

_VMAIN = 999936
_VTAIL = _V - _VMAIN
_LB = 1024
_PB = _LB // 4
_NFULL = _V // _LB
_TAIL = _V - _NFULL * _LB
_PTAIL = _TAIL // 4
_KMAX = (_NFULL + _NW - 1) // _NW


def _transform_body(wt_hbm, wp_hbm, vin, vout):
    wid = lax.axis_index("s") * _NC + lax.axis_index("c")
    iota = lax.iota(jnp.int32, 16)
    d_lo = iota
    d_hi = iota + 16
    zeros = jnp.full((16,), 0, jnp.int32)

    def do_block(l0, L, P):
        l0 = pl.multiple_of(l0, 128)
        pltpu.sync_copy(wt_hbm.at[:, pl.ds(l0, L)], vin.at[:, pl.ds(0, L)])

        @plsc.parallel_loop(0, L, unroll=4)
        def body(e0):
            ev = (zeros + e0 + d_lo) & (L - 1)
            pv = ev >> 2
            jb = (ev & 3) * 32
            for h in range(2):
                dv = d_lo if h == 0 else d_hi
                vals = plsc.load_gather(vin, [dv, ev])
                plsc.store_scatter(vout, [pv, jb + dv], vals)
        pltpu.sync_copy(vout.at[pl.ds(0, P), :],
                        wp_hbm.at[pl.ds(pl.multiple_of(l0 // 4, 8), P), :])

    def step(k, carry):
        b = wid + k * _NW

        @pl.when(b < _NFULL)
        def _():
            do_block(b * _LB, _LB, _PB)
        return carry
    lax.fori_loop(0, _KMAX, step, 0)

    @pl.when(wid == 16)
    def _():
        do_block(_NFULL * _LB, 512, 128)


_CB = _C // _S


def _gather_body(idx_hbm, table_hbm, tail_hbm, out_hbm, idx_v, rows_v, obuf,
                 sem, tail_v):
    wid = lax.axis_index("s") * _NC + lax.axis_index("c")
    base = wid * _RPW
    pltpu.sync_copy(tail_hbm, tail_v)
    iota = lax.iota(jnp.int32, 16)
    iota26 = iota * _S
    for i in range(_NCHUNK):
        off = base + i * _C
        b0 = off // _S
        pltpu.sync_copy(idx_hbm.at[pl.ds(off, _C)], idx_v)
        pltpu.async_copy(table_hbm.at[idx_v], rows_v, sem).wait()

        def fix(v, carry):
            ids = idx_v[pl.ds(v * 16, 16)]
            m = ids >= _VMAIN
            npc = plsc.all_reduce_population_count(m)

            @pl.when(npc[0] > 0)
            def _():
                t = jnp.clip(ids - _VMAIN, 0, _VTAIL - 1)
                rowpos = v * 16 + iota
                for c in range(_D):
                    cvec = jnp.full((16,), 0, jnp.int32) + c
                    vals = plsc.load_gather(tail_v, [t, cvec])
                    plsc.store_scatter(rows_v, [rowpos, cvec], vals, mask=m)
            return carry
        lax.fori_loop(0, _C // 16, fix, 0)

        def rblock(rb, carry):
            r = rb >> 2
            bb0 = (rb & 3) * 16
            jj = iota26 + (bb0 * _S + r)
            bbv = iota + bb0
            rv = jnp.full((16,), 0, jnp.int32) + r

            @plsc.parallel_loop(0, _D, unroll=8)
            def dloop(d0):
                dv = (iota + d0) & (_D - 1)
                vals = plsc.load_gather(rows_v, [jj, dv])
                plsc.store_scatter(obuf, [rv, dv, bbv], vals)
            return carry
        lax.fori_loop(0, _S * 4, rblock, 0)

        for r in range(_S):
            pltpu.sync_copy(obuf.at[r],
                            out_hbm.at[r, :, pl.ds(pl.multiple_of(b0, 8), _CB)])


_MESH = plsc.VectorSubcoreMesh(core_axis_name="c", subcore_axis_name="s")


@jax.jit
def kernel(token_ids, weights):
    wt = weights.T
    transform = pl.kernel(
        _transform_body,
        mesh=_MESH,
        out_type=jax.ShapeDtypeStruct((_V // 4, 128), jnp.float32),
        compiler_params=pltpu.CompilerParams(use_tc_tiling_on_sc=True, needs_layout_passes=False),
        scratch_types=[
            pltpu.VMEM((_D, _LB), jnp.float32),
            pltpu.VMEM((_PB, 128), jnp.float32),
        ],
    )
    table = transform(wt).reshape(_V, _D)
    tail = weights[_VMAIN:, :]
    flat_ids = token_ids.reshape(_ROWS).astype(jnp.int32)
    run = pl.kernel(
        _gather_body,
        mesh=_MESH,
        out_type=jax.ShapeDtypeStruct((_S, _D, _B), jnp.float32),
        compiler_params=pltpu.CompilerParams(use_tc_tiling_on_sc=False, needs_layout_passes=False),
        scratch_types=[
            pltpu.VMEM((_C,), jnp.int32),
            pltpu.VMEM((_C, _D), jnp.float32),
            pltpu.VMEM((_S, _D, _CB), jnp.float32),
            pltpu.SemaphoreType.DMA,
            pltpu.VMEM((_VTAIL, _D), jnp.float32),
        ],
    )
    out = run(flat_ids, table, tail)
    return out.transpose(2, 0, 1)


# speedup vs baseline: 4.2402x; 1.2680x over previous
import jax
import jax.numpy as jnp
from jax import lax
from jax.experimental import pallas as pl
from jax.experimental.pallas import tpu as pltpu
from jax.experimental.pallas import tpu_sc as plsc

_B = 16384
_S = 26
_ROWS = _B * _S
_D = 32
_V = 1000000
_NC = 2
_NS = 16
_NW = _NC * _NS
_RPW = _ROWS // _NW
_NCHUNK = 8
_C = _RPW // _NCHUNK

_VMAIN = 999936
_VTAIL = _V - _VMAIN
_LB = 512
_PB = _LB // 4
_NFULL = _VMAIN // _LB
_KMAX = (_NFULL + _NW - 1) // _NW


def _transform_body(wt_hbm, wp_hbm, vin0, vin1, vout0, vout1,
                    sin0, sin1, sout0, sout1):
    wid = lax.axis_index("s") * _NC + lax.axis_index("c")
    iota = lax.iota(jnp.int32, 16)
    d_lo = iota
    d_hi = iota + 16
    zeros = jnp.full((16,), 0, jnp.int32)
    vins = (vin0, vin1)
    vouts = (vout0, vout1)
    sins = (sin0, sin1)
    souts = (sout0, sout1)

    def in_start(b, buf):
        @pl.when(b < _NFULL)
        def _():
            l0 = pl.multiple_of(b * _LB, 128)
            pltpu.async_copy(wt_hbm.at[:, pl.ds(l0, _LB)], vins[buf],
                             sins[buf])

    def compute(vin, vout):
        @plsc.parallel_loop(0, _LB, unroll=4)
        def body(e0):
            ev = (zeros + e0 + d_lo) & (_LB - 1)
            pv = ev >> 2
            jb = (ev & 3) * 32
            for h in range(2):
                dv = d_lo if h == 0 else d_hi
                vals = plsc.load_gather(vin, [dv, ev])
                plsc.store_scatter(vout, [pv, jb + dv], vals)

    def half(k2, half_idx, buf):
        b = wid + (k2 * 2 + half_idx) * _NW

        @pl.when(b < _NFULL)
        def _():
            pltpu.make_async_copy(wt_hbm.at[:, pl.ds(0, _LB)], vins[buf],
                                  sins[buf]).wait()

            @pl.when(k2 > 0)
            def _():
                pltpu.make_async_copy(vouts[buf],
                                      wp_hbm.at[pl.ds(0, _PB), :],
                                      souts[buf]).wait()
            compute(vins[buf], vouts[buf])
            in_start(b + 2 * _NW, buf)
            pltpu.async_copy(
                vouts[buf],
                wp_hbm.at[pl.ds(pl.multiple_of(b * _PB, 8), _PB), :],
                souts[buf])

    in_start(wid, 0)
    in_start(wid + _NW, 1)

    def pair(k2, carry):
        half(k2, 0, 0)
        half(k2, 1, 1)
        return carry
    lax.fori_loop(0, _KMAX // 2, pair, 0)
    for buf in range(2):
        pltpu.make_async_copy(vouts[buf], wp_hbm.at[pl.ds(0, _PB), :],
                              souts[buf]).wait()


_CB = _C // _S


def _gather_body(idx_hbm, table_hbm, tail_hbm, out_hbm, idx_v, rows_v, obuf,
                 sem, tail_v):
    wid = lax.axis_index("s") * _NC + lax.axis_index("c")
    base = wid * _RPW
    pltpu.sync_copy(tail_hbm, tail_v)
    iota = lax.iota(jnp.int32, 16)
    iota26 = iota * _S
    for i in range(_NCHUNK):
        off = base + i * _C
        b0 = off // _S
        pltpu.sync_copy(idx_hbm.at[pl.ds(off, _C)], idx_v)
        pltpu.async_copy(table_hbm.at[idx_v], rows_v, sem).wait()

        def fix(v, carry):
            ids = idx_v[pl.ds(v * 16, 16)]
            m = ids >= _VMAIN
            npc = plsc.all_reduce_population_count(m)

            @pl.when(npc[0] > 0)
            def _():
                t = jnp.clip(ids - _VMAIN, 0, _VTAIL - 1)
                rowpos = v * 16 + iota
                for c in range(_D):
                    cvec = jnp.full((16,), 0, jnp.int32) + c
                    vals = plsc.load_gather(tail_v, [t, cvec])
                    plsc.store_scatter(rows_v, [rowpos, cvec], vals, mask=m)
            return carry
        lax.fori_loop(0, _C // 16, fix, 0)

        def rblock(rb, carry):
            r = rb >> 2
            bb0 = (rb & 3) * 16
            jj = iota26 + (bb0 * _S + r)
            bbv = iota + bb0
            rv = jnp.full((16,), 0, jnp.int32) + r

            @plsc.parallel_loop(0, _D, unroll=8)
            def dloop(d0):
                dv = (iota + d0) & (_D - 1)
                vals = plsc.load_gather(rows_v, [jj, dv])
                plsc.store_scatter(obuf, [rv, dv, bbv], vals)
            return carry
        lax.fori_loop(0, _S * 4, rblock, 0)

        for r in range(_S):
            pltpu.sync_copy(obuf.at[r],
                            out_hbm.at[r, :, pl.ds(pl.multiple_of(b0, 8), _CB)])


_MESH = plsc.VectorSubcoreMesh(core_axis_name="c", subcore_axis_name="s")


@jax.jit
def kernel(token_ids, weights):
    wt = weights.T
    transform = pl.kernel(
        _transform_body,
        mesh=_MESH,
        out_type=jax.ShapeDtypeStruct((_V // 4, 128), jnp.float32),
        compiler_params=pltpu.CompilerParams(use_tc_tiling_on_sc=True, needs_layout_passes=False),
        scratch_types=[
            pltpu.VMEM((_D, _LB), jnp.float32),
            pltpu.VMEM((_D, _LB), jnp.float32),
            pltpu.VMEM((_PB, 128), jnp.float32),
            pltpu.VMEM((_PB, 128), jnp.float32),
            pltpu.SemaphoreType.DMA,
            pltpu.SemaphoreType.DMA,
            pltpu.SemaphoreType.DMA,
            pltpu.SemaphoreType.DMA,
        ],
    )
    table = transform(wt).reshape(_V, _D)
    tail = weights[_VMAIN:, :]
    flat_ids = token_ids.reshape(_ROWS).astype(jnp.int32)
    run = pl.kernel(
        _gather_body,
        mesh=_MESH,
        out_type=jax.ShapeDtypeStruct((_S, _D, _B), jnp.float32),
        compiler_params=pltpu.CompilerParams(use_tc_tiling_on_sc=False, needs_layout_passes=False),
        scratch_types=[
            pltpu.VMEM((_C,), jnp.int32),
            pltpu.VMEM((_C, _D), jnp.float32),
            pltpu.VMEM((_S, _D, _CB), jnp.float32),
            pltpu.SemaphoreType.DMA,
            pltpu.VMEM((_VTAIL, _D), jnp.float32),
        ],
    )
    out = run(flat_ids, table, tail)
    return out.transpose(2, 0, 1)


# speedup vs baseline: 4.7799x; 1.1273x over previous
import jax
import jax.numpy as jnp
from jax import lax
from jax.experimental import pallas as pl
from jax.experimental.pallas import tpu as pltpu
from jax.experimental.pallas import tpu_sc as plsc

_B = 16384
_S = 26
_ROWS = _B * _S
_D = 32
_V = 1000000
_NC = 2
_NS = 16
_NW = _NC * _NS
_RPW = _ROWS // _NW
_NCHUNK = 8
_C = _RPW // _NCHUNK

_VMAIN = 999936
_VTAIL = _V - _VMAIN
_LB = 512
_PB = _LB // 4
_NFULL = _VMAIN // _LB
_KMAX = (_NFULL + _NW - 1) // _NW


def _transform_body(wt_hbm, wp_hbm, vin0, vin1, vout0, vout1,
                    sin0, sin1, sout0, sout1):
    wid = lax.axis_index("s") * _NC + lax.axis_index("c")
    iota = lax.iota(jnp.int32, 16)
    d_lo = iota
    d_hi = iota + 16
    zeros = jnp.full((16,), 0, jnp.int32)
    vins = (vin0, vin1)
    vouts = (vout0, vout1)
    sins = (sin0, sin1)
    souts = (sout0, sout1)

    def in_start(b, buf):
        @pl.when(b < _NFULL)
        def _():
            l0 = pl.multiple_of(b * _LB, 128)
            pltpu.async_copy(wt_hbm.at[:, pl.ds(l0, _LB)], vins[buf],
                             sins[buf])

    def compute(vin, vout):
        @plsc.parallel_loop(0, _LB, unroll=4)
        def body(e0):
            ev = (zeros + e0 + d_lo) & (_LB - 1)
            pv = ev >> 2
            jb = (ev & 3) * 32
            for h in range(2):
                dv = d_lo if h == 0 else d_hi
                vals = plsc.load_gather(vin, [dv, ev])
                plsc.store_scatter(vout, [pv, jb + dv], vals)

    def half(k2, half_idx, buf):
        b = wid + (k2 * 2 + half_idx) * _NW

        @pl.when(b < _NFULL)
        def _():
            pltpu.make_async_copy(wt_hbm.at[:, pl.ds(0, _LB)], vins[buf],
                                  sins[buf]).wait()

            @pl.when(k2 > 0)
            def _():
                pltpu.make_async_copy(vouts[buf],
                                      wp_hbm.at[pl.ds(0, _PB), :],
                                      souts[buf]).wait()
            compute(vins[buf], vouts[buf])
            in_start(b + 2 * _NW, buf)
            pltpu.async_copy(
                vouts[buf],
                wp_hbm.at[pl.ds(pl.multiple_of(b * _PB, 8), _PB), :],
                souts[buf])

    in_start(wid, 0)
    in_start(wid + _NW, 1)

    def pair(k2, carry):
        half(k2, 0, 0)
        half(k2, 1, 1)
        return carry
    lax.fori_loop(0, _KMAX // 2, pair, 0)
    for buf in range(2):
        pltpu.make_async_copy(vouts[buf], wp_hbm.at[pl.ds(0, _PB), :],
                              souts[buf]).wait()


_NCH2 = 16
_C2 = _RPW // _NCH2
_CB = _C2 // _S


def _gather_body(idx_hbm, table_hbm, tail_hbm, out_hbm,
                 idx0, idx1, rows0, rows1, obuf, tail_v,
                 sg0, sg1, so):
    wid = lax.axis_index("s") * _NC + lax.axis_index("c")
    base = wid * _RPW
    pltpu.sync_copy(tail_hbm, tail_v)
    iota = lax.iota(jnp.int32, 16)
    iota26 = iota * _S
    idxs = (idx0, idx1)
    rows = (rows0, rows1)
    sgs = (sg0, sg1)

    def gstart(i, buf):
        off = base + i * _C2
        pltpu.sync_copy(idx_hbm.at[pl.ds(off, _C2)], idxs[buf])
        pltpu.async_copy(table_hbm.at[idxs[buf]], rows[buf], sgs[buf])

    gstart(0, 0)
    gstart(1, 1)
    for i in range(_NCH2):
        buf = i & 1
        idx_v = idxs[buf]
        rows_v = rows[buf]
        b0 = (base + i * _C2) // _S
        pltpu.make_async_copy(table_hbm.at[idx_v], rows_v, sgs[buf]).wait()

        def fix(v, carry):
            ids = idx_v[pl.ds(v * 16, 16)]
            m = ids >= _VMAIN
            npc = plsc.all_reduce_population_count(m)

            @pl.when(npc[0] > 0)
            def _():
                t = jnp.clip(ids - _VMAIN, 0, _VTAIL - 1)
                rowpos = v * 16 + iota
                for c in range(_D):
                    cvec = jnp.full((16,), 0, jnp.int32) + c
                    vals = plsc.load_gather(tail_v, [t, cvec])
                    plsc.store_scatter(rows_v, [rowpos, cvec], vals, mask=m)
            return carry
        lax.fori_loop(0, _C2 // 16, fix, 0)

        if i > 0:
            pb0 = (base + (i - 1) * _C2) // _S
            for r in range(_S):
                pltpu.make_async_copy(
                    obuf.at[r],
                    out_hbm.at[r, :, pl.ds(pl.multiple_of(pb0, 8), _CB)],
                    so).wait()

        def rblock(rb, carry):
            r = rb >> 1
            bb0 = (rb & 1) * 16
            jj = iota26 + (bb0 * _S + r)
            bbv = iota + bb0
            rv = jnp.full((16,), 0, jnp.int32) + r

            @plsc.parallel_loop(0, _D, unroll=8)
            def dloop(d0):
                dv = (iota + d0) & (_D - 1)
                vals = plsc.load_gather(rows_v, [jj, dv])
                plsc.store_scatter(obuf, [rv, dv, bbv], vals)
            return carry
        lax.fori_loop(0, _S * (_CB // 16), rblock, 0)

        if i + 2 < _NCH2:
            gstart(i + 2, buf)
        for r in range(_S):
            pltpu.async_copy(obuf.at[r],
                             out_hbm.at[r, :, pl.ds(pl.multiple_of(b0, 8), _CB)],
                             so)
    fb0 = (base + (_NCH2 - 1) * _C2) // _S
    for r in range(_S):
        pltpu.make_async_copy(
            obuf.at[r],
            out_hbm.at[r, :, pl.ds(pl.multiple_of(fb0, 8), _CB)],
            so).wait()


_MESH = plsc.VectorSubcoreMesh(core_axis_name="c", subcore_axis_name="s")


@jax.jit
def kernel(token_ids, weights):
    wt = weights.T
    transform = pl.kernel(
        _transform_body,
        mesh=_MESH,
        out_type=jax.ShapeDtypeStruct((_V // 4, 128), jnp.float32),
        compiler_params=pltpu.CompilerParams(use_tc_tiling_on_sc=True, needs_layout_passes=False),
        scratch_types=[
            pltpu.VMEM((_D, _LB), jnp.float32),
            pltpu.VMEM((_D, _LB), jnp.float32),
            pltpu.VMEM((_PB, 128), jnp.float32),
            pltpu.VMEM((_PB, 128), jnp.float32),
            pltpu.SemaphoreType.DMA,
            pltpu.SemaphoreType.DMA,
            pltpu.SemaphoreType.DMA,
            pltpu.SemaphoreType.DMA,
        ],
    )
    table = transform(wt).reshape(_V, _D)
    tail = weights[_VMAIN:, :]
    flat_ids = token_ids.reshape(_ROWS).astype(jnp.int32)
    run = pl.kernel(
        _gather_body,
        mesh=_MESH,
        out_type=jax.ShapeDtypeStruct((_S, _D, _B), jnp.float32),
        compiler_params=pltpu.CompilerParams(use_tc_tiling_on_sc=False, needs_layout_passes=False),
        scratch_types=[
            pltpu.VMEM((_C2,), jnp.int32),
            pltpu.VMEM((_C2,), jnp.int32),
            pltpu.VMEM((_C2, _D), jnp.float32),
            pltpu.VMEM((_C2, _D), jnp.float32),
            pltpu.VMEM((_S, _D, _CB), jnp.float32),
            pltpu.VMEM((_VTAIL, _D), jnp.float32),
            pltpu.SemaphoreType.DMA,
            pltpu.SemaphoreType.DMA,
            pltpu.SemaphoreType.DMA,
        ],
    )
    out = run(flat_ids, table, tail)
    return out.transpose(2, 0, 1)
